# baseline (speedup 1.0000x reference)
import jax
import jax.numpy as jnp
from jax.experimental import pallas as pl
from jax.experimental.pallas import tpu as pltpu

BATCH = 16384
FIELDS = 4
EMBED = 64
BLK = 2048


def _fm_body(first_ref, second_ref, bias_ref, out_ref):
    x0 = second_ref[:, 0, :]
    x1 = second_ref[:, 1, :]
    x2 = second_ref[:, 2, :]
    x3 = second_ref[:, 3, :]
    s = (x0 + x1) + (x2 + x3)
    sq = x0 * x0 + x1 * x1 + x2 * x2 + x3 * x3
    inter = jnp.sum(s * s - sq, axis=1, keepdims=True)
    ft = jnp.sum(first_ref[...], axis=1, keepdims=True)
    out_ref[...] = bias_ref[0, 0] + ft + 0.5 * inter


def kernel(first_embeddings, second_embeddings, bias):
    b2 = bias.reshape(1, 1)
    out = pl.pallas_call(
        _fm_body,
        grid=(BATCH // BLK,),
        in_specs=[
            pl.BlockSpec((BLK, FIELDS), lambda i: (i, 0)),
            pl.BlockSpec((BLK, FIELDS, EMBED), lambda i: (i, 0, 0)),
            pl.BlockSpec((1, 1), lambda i: (0, 0)),
        ],
        out_specs=pl.BlockSpec((BLK, 1), lambda i: (i, 0)),
        out_shape=jax.ShapeDtypeStruct((BATCH, 1), jnp.float32),
    )(first_embeddings, second_embeddings, b2)
    return out.reshape(BATCH)


# speedup vs baseline: 7.3395x; 7.3395x over previous
import jax
import jax.numpy as jnp
from jax.experimental import pallas as pl
from jax.experimental.pallas import tpu as pltpu

BATCH = 16384
FIELDS = 4
EMBED = 64
CBLK = 2048


def _fm_body(first_ref, second_ref, bias_ref, out_ref):
    x = second_ref[...]
    q = x * x
    s = (x[0:64, :] + x[64:128, :]) + (x[128:192, :] + x[192:256, :])
    sq = (q[0:64, :] + q[64:128, :]) + (q[128:192, :] + q[192:256, :])
    t = s * s - sq
    inter = jnp.sum(t, axis=0, keepdims=True)
    ft = jnp.sum(first_ref[...], axis=0, keepdims=True)
    out_ref[...] = bias_ref[0, 0] + ft + 0.5 * inter


def kernel(first_embeddings, second_embeddings, bias):
    xt = jnp.transpose(second_embeddings, (1, 2, 0)).reshape(FIELDS * EMBED, BATCH)
    ft = jnp.transpose(first_embeddings, (1, 0))
    b2 = bias.reshape(1, 1)
    out = pl.pallas_call(
        _fm_body,
        grid=(BATCH // CBLK,),
        in_specs=[
            pl.BlockSpec((FIELDS, CBLK), lambda j: (0, j)),
            pl.BlockSpec((FIELDS * EMBED, CBLK), lambda j: (0, j)),
            pl.BlockSpec((1, 1), lambda j: (0, 0)),
        ],
        out_specs=pl.BlockSpec((1, CBLK), lambda j: (0, j)),
        out_shape=jax.ShapeDtypeStruct((1, BATCH), jnp.float32),
    )(ft, xt, b2)
    return out.reshape(BATCH)
